# baseline (speedup 1.0000x reference)
import functools

import jax
import jax.numpy as jnp
from jax import lax
from jax.experimental import pallas as pl
from jax.experimental.pallas import tpu as pltpu
from jax.experimental.pallas import tpu_sc as plsc

N = 10000
E = 160000
DL = 128
D_EDGE = 16

SLABS = ((0, 81920), (81920, 78080))
BE = 1280
BN = 2000
GN = N // BN

NC = 2
NS = 16
NW = NC * NS
LANES = 16
CH = 128
NT = 640
NPAD = NT * NS


def _ln(y, scale, offset):
    mu = jnp.mean(y, axis=-1, keepdims=True)
    var = jnp.mean(jnp.square(y - mu), axis=-1, keepdims=True)
    return (y - mu) * lax.rsqrt(var + 1e-5) * scale + offset


def _dot(a, b):
    return jnp.dot(a, b, preferred_element_type=jnp.float32)


def _embed_node_body(x_ref, wen, ben, ws, wr, gf, weg, beg, wge, bge,
                     n0, ns, nr, g0, gb):
    n0v = _dot(x_ref[...], wen[...]) + ben[...]
    n0[...] = n0v
    ns[...] = _dot(n0v, ws[...])
    nr[...] = _dot(n0v, wr[...])
    g0v = _dot(gf[...], weg[...]) + beg[...]
    g0[...] = g0v
    gb[...] = _dot(g0v, wge[...]) + bge[...]


def _embed_nodes(nodes, wen, ben, ws1, wr1, gf, weg, beg, wge1, bge1):
    full = lambda i: (0, 0)
    return pl.pallas_call(
        _embed_node_body,
        grid=(GN,),
        in_specs=[
            pl.BlockSpec((BN, DL), lambda i: (i, 0)),
            pl.BlockSpec((DL, DL), full),
            pl.BlockSpec((1, DL), full),
            pl.BlockSpec((DL, DL), full),
            pl.BlockSpec((DL, DL), full),
            pl.BlockSpec((1, DL), full),
            pl.BlockSpec((DL, DL), full),
            pl.BlockSpec((1, DL), full),
            pl.BlockSpec((DL, DL), full),
            pl.BlockSpec((1, DL), full),
        ],
        out_specs=[
            pl.BlockSpec((BN, DL), lambda i: (i, 0)),
            pl.BlockSpec((BN, DL), lambda i: (i, 0)),
            pl.BlockSpec((BN, DL), lambda i: (i, 0)),
            pl.BlockSpec((1, DL), full),
            pl.BlockSpec((1, DL), full),
        ],
        out_shape=[
            jax.ShapeDtypeStruct((N, DL), jnp.float32),
            jax.ShapeDtypeStruct((N, DL), jnp.float32),
            jax.ShapeDtypeStruct((N, DL), jnp.float32),
            jax.ShapeDtypeStruct((1, DL), jnp.float32),
            jax.ShapeDtypeStruct((1, DL), jnp.float32),
        ],
    )(nodes, wen, ben.reshape(1, DL), ws1, wr1, gf, weg, beg.reshape(1, DL),
      wge1, bge1.reshape(1, DL))


def _edge_common(i, ev, x, lns, lno, new_e, e2, eagg):
    ne = jnp.maximum(x, 0.0)
    new_e[...] = ne
    e2[...] = _ln(ne + ev, lns[...], lno[...])
    part = jnp.sum(ne.reshape(BE // 8, 8, DL), axis=0)

    @pl.when(i == 0)
    def _():
        eagg[...] = part

    @pl.when(i > 0)
    def _():
        eagg[...] = eagg[...] + part


def _edge_embed_body(x_ref, gs_ref, dn_ref, wemb, bemb, we, geb, lns, lno,
                     new_e, e2, eagg):
    del dn_ref
    i = pl.program_id(0)
    ev = _dot(x_ref[...], wemb[...]) + bemb[...]
    x = _dot(ev, we[...]) + gs_ref[...] + geb[...]
    _edge_common(i, ev, x, lns, lno, new_e, e2, eagg)


def _tc_edge_embed(edges, gsum, donor, slab, wemb, bemb, we_e, geb,
                   ln_scale, ln_offset):
    base, size = slab
    grid = size // BE
    off = base // BE
    full = lambda i: (0, 0)
    sblk = lambda i: (i, 0)
    oblk = lambda i: (i + off, 0)
    return pl.pallas_call(
        _edge_embed_body,
        grid=(grid,),
        in_specs=[
            pl.BlockSpec((BE, D_EDGE), oblk),
            pl.BlockSpec((BE, DL), sblk),
            pl.BlockSpec((BE, DL), oblk),
            pl.BlockSpec((D_EDGE, DL), full),
            pl.BlockSpec((1, DL), full),
            pl.BlockSpec((DL, DL), full),
            pl.BlockSpec((1, DL), full),
            pl.BlockSpec((1, DL), full),
            pl.BlockSpec((1, DL), full),
        ],
        out_specs=[
            pl.BlockSpec((BE, DL), sblk),
            pl.BlockSpec((BE, DL), oblk),
            pl.BlockSpec((8, DL), full),
        ],
        out_shape=[
            jax.ShapeDtypeStruct((size, DL), jnp.float32),
            jax.ShapeDtypeStruct((E, DL), jnp.float32),
            jax.ShapeDtypeStruct((8, DL), jnp.float32),
        ],
        input_output_aliases={2: 1},
    )(edges, gsum, donor, wemb, bemb.reshape(1, DL), we_e, geb,
      ln_scale.reshape(1, DL), ln_offset.reshape(1, DL))


def _edge_body(e_ref, gs_ref, we, geb, lns, lno, new_e, e2, eagg):
    i = pl.program_id(0)
    ev = e_ref[...]
    x = _dot(ev, we[...]) + gs_ref[...] + geb[...]
    _edge_common(i, ev, x, lns, lno, new_e, e2, eagg)


def _tc_edge(e_full, gsum, slab, we_e, geb, ln_scale, ln_offset):
    base, size = slab
    grid = size // BE
    off = base // BE
    full = lambda i: (0, 0)
    sblk = lambda i: (i, 0)
    oblk = lambda i: (i + off, 0)
    return pl.pallas_call(
        _edge_body,
        grid=(grid,),
        in_specs=[
            pl.BlockSpec((BE, DL), oblk),
            pl.BlockSpec((BE, DL), sblk),
            pl.BlockSpec((DL, DL), full),
            pl.BlockSpec((1, DL), full),
            pl.BlockSpec((1, DL), full),
            pl.BlockSpec((1, DL), full),
        ],
        out_specs=[
            pl.BlockSpec((BE, DL), sblk),
            pl.BlockSpec((BE, DL), oblk),
            pl.BlockSpec((8, DL), full),
        ],
        out_shape=[
            jax.ShapeDtypeStruct((size, DL), jnp.float32),
            jax.ShapeDtypeStruct((E, DL), jnp.float32),
            jax.ShapeDtypeStruct((8, DL), jnp.float32),
        ],
        input_output_aliases={0: 1},
    )(e_full, gsum, we_e, geb, ln_scale.reshape(1, DL),
      ln_offset.reshape(1, DL))


def _make_node_body(last):
    def body(n_ref, saa, sab, raa, rab, g_ref, eagg_a, eagg_b,
             wn_n, wn_s, wn_r, wn_g, bn, wg_n, wg_e, wg_g, bg,
             lnn_s, lnn_o, lng_s, lng_o, xa, xb, xc, xd,
             n2, o1, o2, o3, o4, nacc):
        i = pl.program_id(0)
        nv = n_ref[...]
        gv = g_ref[...]
        sa = saa[...] + sab[...]
        ra = raa[...] + rab[...]
        gn = _dot(gv, wn_g[...]) + bn[...]
        x = _dot(nv, wn_n[...]) + _dot(sa, wn_s[...]) \
            + _dot(ra, wn_r[...]) + gn
        nn = jnp.maximum(x, 0.0)
        n2v = _ln(nn + nv, lnn_s[...], lnn_o[...])
        n2[...] = n2v
        part = jnp.sum(nn.reshape(BN // 8, 8, DL), axis=0)

        @pl.when(i == 0)
        def _():
            nacc[...] = part

        @pl.when(i > 0)
        def _():
            nacc[...] = nacc[...] + part

        if not last:
            o1[...] = _dot(n2v, xa[...])
            o2[...] = _dot(n2v, xb[...])

        @pl.when(i == GN - 1)
        def _():
            nagg = jnp.sum(nacc[...], axis=0, keepdims=True)
            eagg = jnp.sum(eagg_a[...] + eagg_b[...], axis=0, keepdims=True)
            ng = jnp.maximum(
                _dot(nagg, wg_n[...]) + _dot(eagg, wg_e[...])
                + _dot(gv, wg_g[...]) + bg[...], 0.0)
            g2v = _ln(ng + gv, lng_s[...], lng_o[...])
            if last:
                o1[...] = _dot(g2v, xa[...]) + xb[...]
            else:
                o3[...] = g2v
                o4[...] = _dot(g2v, xc[...]) + xd[...]

    return body


def _tc_node(n, agg_a, agg_b, g, eagg_a, eagg_b, sp, last, xa, xb,
             xc=None, xd=None):
    wn = sp["node_mlp"]["w"]
    wg = sp["global_mlp"]["w"]
    full = lambda i: (0, 0)
    blk = lambda i: (i, 0)
    rblk = lambda i: (i + N // BN, 0)
    row = pl.BlockSpec((1, DL), full)
    mat = pl.BlockSpec((DL, DL), full)
    nblk = pl.BlockSpec((BN, DL), blk)
    agg8 = pl.BlockSpec((8, DL), full)
    in_specs = [nblk,
                pl.BlockSpec((BN, DL), blk), pl.BlockSpec((BN, DL), blk),
                pl.BlockSpec((BN, DL), rblk), pl.BlockSpec((BN, DL), rblk),
                row, agg8, agg8,
                mat, mat, mat, mat, row, mat, mat, mat, row,
                row, row, row, row]
    args = [n, agg_a, agg_b, agg_a, agg_b, g, eagg_a, eagg_b,
            wn[:DL], wn[DL:2 * DL], wn[2 * DL:3 * DL], wn[3 * DL:],
            sp["node_mlp"]["b"].reshape(1, DL),
            wg[:DL], wg[DL:2 * DL], wg[2 * DL:],
            sp["global_mlp"]["b"].reshape(1, DL),
            sp["ln_nodes"]["scale"].reshape(1, DL),
            sp["ln_nodes"]["offset"].reshape(1, DL),
            sp["ln_globals"]["scale"].reshape(1, DL),
            sp["ln_globals"]["offset"].reshape(1, DL)]
    if last:
        in_specs += [mat, row]
        args += [xa, xb.reshape(1, DL)]
        out_specs = [nblk, row]
        out_shape = [jax.ShapeDtypeStruct((N, DL), jnp.float32),
                     jax.ShapeDtypeStruct((1, DL), jnp.float32)]
    else:
        in_specs += [mat, mat, mat, row]
        args += [xa, xb, xc, xd.reshape(1, DL)]
        out_specs = [nblk, nblk, nblk, row, row]
        out_shape = [jax.ShapeDtypeStruct((N, DL), jnp.float32),
                     jax.ShapeDtypeStruct((N, DL), jnp.float32),
                     jax.ShapeDtypeStruct((N, DL), jnp.float32),
                     jax.ShapeDtypeStruct((1, DL), jnp.float32),
                     jax.ShapeDtypeStruct((1, DL), jnp.float32)]
    body = _make_node_body(last)
    nin = len(in_specs)
    nout = len(out_specs)
    if last:
        def wrapped(*refs):
            ins = refs[:nin]
            n2, o1 = refs[nin:nin + nout]
            nacc = refs[nin + nout]
            body(*ins[:21], ins[21], ins[22], None, None,
                 n2, o1, None, None, None, nacc)
    else:
        def wrapped(*refs):
            ins = refs[:nin]
            n2, o1, o2, o3, o4 = refs[nin:nin + nout]
            nacc = refs[nin + nout]
            body(*ins[:21], ins[21], ins[22], ins[23], ins[24],
                 n2, o1, o2, o3, o4, nacc)
    return pl.pallas_call(
        wrapped,
        grid=(GN,),
        in_specs=in_specs,
        out_specs=out_specs,
        out_shape=out_shape,
        scratch_shapes=[pltpu.VMEM((8, DL), jnp.float32)],
    )(*args)


def _sc_gather(ns, nr, senders, receivers, slab):
    base_g, size = slab
    sw = size // NW
    nchunk = -(-sw // CH)
    assert nchunk % 2 == 0 and sw >= CH
    mesh = plsc.VectorSubcoreMesh(core_axis_name="c", subcore_axis_name="s")

    @functools.partial(
        pl.kernel,
        mesh=mesh,
        out_type=jax.ShapeDtypeStruct((size, DL), jnp.float32),
        scratch_types=[
            pltpu.VMEM((sw,), jnp.int32),
            pltpu.VMEM((sw,), jnp.int32),
            pltpu.VMEM((2, CH, DL), jnp.float32),
            pltpu.VMEM((2, CH, DL), jnp.float32),
            pltpu.VMEM((2, CH, DL), jnp.float32),
            pltpu.SemaphoreType.DMA,
            pltpu.SemaphoreType.DMA,
        ],
    )
    def k(ns_h, nr_h, s_h, r_h, out_h, idx_s, idx_r, rows_a, rows_b,
          rows_o, sem0, sem1):
        wid = lax.axis_index("s") * NC + lax.axis_index("c")
        lbase = wid * sw
        sems = (sem0, sem1)

        pltpu.sync_copy(s_h.at[pl.ds(base_g + lbase, sw)], idx_s)
        pltpu.sync_copy(r_h.at[pl.ds(base_g + lbase, sw)], idx_r)

        def loff(c):
            return jnp.minimum(c * CH, sw - CH)

        def issue(c, b):
            o = loff(c)
            pltpu.async_copy(ns_h.at[idx_s.at[pl.ds(o, CH)]],
                             rows_a.at[b], sems[b])
            pltpu.async_copy(nr_h.at[idx_r.at[pl.ds(o, CH)]],
                             rows_b.at[b], sems[b])

        def drain(c, b):
            o = loff(c)
            pltpu.make_async_copy(ns_h.at[idx_s.at[pl.ds(o, CH)]],
                                  rows_a.at[b], sems[b]).wait()
            pltpu.make_async_copy(nr_h.at[idx_r.at[pl.ds(o, CH)]],
                                  rows_b.at[b], sems[b]).wait()

        def process(c, b):
            def add_row(i, carry):
                for v in range(DL // LANES):
                    sl = pl.ds(v * LANES, LANES)
                    rows_o[b, i, sl] = rows_a[b, i, sl] + rows_b[b, i, sl]
                return carry

            lax.fori_loop(0, CH, add_row, 0)
            pltpu.sync_copy(rows_o.at[b], out_h.at[pl.ds(lbase + loff(c), CH)])

        issue(0, 0)

        def body(j2, _):
            c0 = j2 * 2
            issue(c0 + 1, 1)
            drain(c0, 0)
            process(c0, 0)
            issue(c0 + 2, 0)
            drain(c0 + 1, 1)
            process(c0 + 1, 1)
            return 0

        lax.fori_loop(0, nchunk // 2, body, 0)
        drain(nchunk, 0)

    return k(ns, nr, senders, receivers)


def _sc_scatter(new_e, sr_flat, zrows, slab):
    base_g, size = slab
    ts = size // NS
    nchunk = ts // CH
    tail = ts - nchunk * CH
    assert nchunk % 2 == 0 and tail % 8 == 0
    mesh = plsc.VectorSubcoreMesh(core_axis_name="c", subcore_axis_name="s")

    @functools.partial(
        pl.kernel,
        mesh=mesh,
        out_type=jax.ShapeDtypeStruct((2 * N, DL), jnp.float32),
        scratch_types=[
            pltpu.VMEM((nchunk, CH), jnp.int32),
            pltpu.VMEM((LANES,), jnp.int32),
            pltpu.VMEM((2, CH, DL), jnp.float32),
            pltpu.VMEM_SHARED((NPAD, DL), jnp.float32),
            pltpu.SemaphoreType.DMA,
            pltpu.SemaphoreType.DMA,
        ],
    )
    def k(ne_h, sr_h, z_h, out_h, idx2d, idx_t, rows, acc, sem0, sem1):
        c = lax.axis_index("c")
        s = lax.axis_index("s")
        lbase = s * ts
        gbase = c * E + base_g + lbase
        sems = (sem0, sem1)
        pltpu.sync_copy(z_h, acc.at[pl.ds(s * NT, NT)])
        plsc.subcore_barrier()

        def issue(j, b):
            jc = jnp.minimum(j, nchunk - 1)
            pltpu.async_copy(sr_h.at[pl.ds(gbase + jc * CH, CH)],
                             idx2d.at[jc], sems[b])
            pltpu.async_copy(ne_h.at[pl.ds(lbase + jc * CH, CH)],
                             rows.at[b], sems[b])

        def drain(j, b):
            jc = jnp.minimum(j, nchunk - 1)
            pltpu.make_async_copy(sr_h.at[pl.ds(gbase + jc * CH, CH)],
                                  idx2d.at[jc], sems[b]).wait()
            pltpu.make_async_copy(ne_h.at[pl.ds(lbase + jc * CH, CH)],
                                  rows.at[b], sems[b]).wait()

        def scat(j, b):
            pltpu.sync_copy(rows.at[b],
                            acc.at[idx2d.at[jnp.minimum(j, nchunk - 1)]],
                            add=True)

        issue(0, 0)

        def body(j2, _):
            c0 = j2 * 2
            issue(c0 + 1, 1)
            drain(c0, 0)
            scat(c0, 0)
            issue(c0 + 2, 0)
            drain(c0 + 1, 1)
            scat(c0 + 1, 1)
            return 0

        lax.fori_loop(0, nchunk // 2, body, 0)
        drain(nchunk, 0)
        if tail:
            off = lbase + nchunk * CH
            pltpu.sync_copy(sr_h.at[pl.ds(c * E + base_g + off, tail)], idx_t)
            pltpu.sync_copy(ne_h.at[pl.ds(off, tail)],
                            rows.at[0, pl.ds(0, tail)])
            pltpu.sync_copy(rows.at[0, pl.ds(0, tail)], acc.at[idx_t],
                            add=True)
        plsc.subcore_barrier()

        @pl.when(s < NS - 1)
        def _():
            pltpu.sync_copy(acc.at[pl.ds(s * NT, NT)],
                            out_h.at[pl.ds(c * N + s * NT, NT)])

        @pl.when(s == NS - 1)
        def _():
            pltpu.sync_copy(acc.at[pl.ds((NS - 1) * NT, N - (NS - 1) * NT)],
                            out_h.at[pl.ds(c * N + (NS - 1) * NT,
                                           N - (NS - 1) * NT)])

    return k(new_e, sr_flat, zrows)


def kernel(nodes, edges, senders, receivers, global_feats, params):
    p = params
    st = p["steps"]

    def edge_w(s):
        w = st[s]["edge_mlp"]["w"]
        return w[:DL], w[DL:2 * DL], w[2 * DL:3 * DL], w[3 * DL:]

    sr_flat = jnp.concatenate([senders, receivers])
    zrows = jnp.zeros((NT, DL), jnp.float32)
    e_donor = jnp.zeros((E, DL), jnp.float32)

    we_e0, we_s0, we_r0, we_g0 = edge_w(0)
    n, ns, nr, g, gb = _embed_nodes(
        nodes, p["embed_node"]["w"], p["embed_node"]["b"], we_s0, we_r0,
        global_feats, p["embed_global"]["w"], p["embed_global"]["b"],
        we_g0, st[0]["edge_mlp"]["b"])

    e_full = e_donor
    gdec = None
    for s in range(len(st)):
        last = s == len(st) - 1
        lnsc = st[s]["ln_edges"]["scale"]
        lnof = st[s]["ln_edges"]["offset"]
        new_es, eaggs, aggs = [], [], []
        gsums = [_sc_gather(ns, nr, senders, receivers, sl) for sl in SLABS]
        for sl, gsum in zip(SLABS, gsums):
            if s == 0:
                ne_sl, e_full, eagg = _tc_edge_embed(
                    edges, gsum, e_full, sl, p["embed_edge"]["w"],
                    p["embed_edge"]["b"], we_e0, gb, lnsc, lnof)
            else:
                ne_sl, e_full, eagg = _tc_edge(
                    e_full, gsum, sl, edge_w(s)[0], gb, lnsc, lnof)
            new_es.append(ne_sl)
            eaggs.append(eagg)
            aggs.append(_sc_scatter(ne_sl, sr_flat, zrows, sl))
        if last:
            n, gdec = _tc_node(n, aggs[0], aggs[1], g, eaggs[0], eaggs[1],
                               st[s], True, p["decoder"]["w"],
                               p["decoder"]["b"])
        else:
            _, we_s1, we_r1, we_g1 = edge_w(s + 1)
            n, ns, nr, g, gb = _tc_node(n, aggs[0], aggs[1], g, eaggs[0],
                                        eaggs[1], st[s], False,
                                        we_s1, we_r1, we_g1,
                                        st[s + 1]["edge_mlp"]["b"])
    return (n, e_full, gdec)


# speedup vs baseline: 1.0586x; 1.0586x over previous
import functools

import jax
import jax.numpy as jnp
from jax import lax
from jax.experimental import pallas as pl
from jax.experimental.pallas import tpu as pltpu
from jax.experimental.pallas import tpu_sc as plsc

N = 10000
E = 160000
DL = 128
D_EDGE = 16

BE = 2000
GE = E // BE
BN = 2000
GN = N // BN

NC = 2
NS = 16
NW = NC * NS
LANES = 16
CH = 128
EW = E // NW
ET = E // NS
NT = 640
NPAD = NT * NS


def _ln(y, scale, offset):
    mu = jnp.mean(y, axis=-1, keepdims=True)
    var = jnp.mean(jnp.square(y - mu), axis=-1, keepdims=True)
    return (y - mu) * lax.rsqrt(var + 1e-5) * scale + offset


def _dot(a, b):
    return jnp.dot(a, b, preferred_element_type=jnp.float32)


def _embed_edge_body(x_ref, w_ref, b_ref, o_ref):
    o_ref[...] = _dot(x_ref[...], w_ref[...]) + b_ref[...]


def _embed_edges(edges, w, b):
    return pl.pallas_call(
        _embed_edge_body,
        grid=(GE,),
        in_specs=[
            pl.BlockSpec((BE, D_EDGE), lambda i: (i, 0)),
            pl.BlockSpec((D_EDGE, DL), lambda i: (0, 0)),
            pl.BlockSpec((1, DL), lambda i: (0, 0)),
        ],
        out_specs=pl.BlockSpec((BE, DL), lambda i: (i, 0)),
        out_shape=jax.ShapeDtypeStruct((E, DL), jnp.float32),
    )(edges, w, b.reshape(1, DL))


def _embed_node_body(x_ref, wen, ben, ws, wr, gf, weg, beg, wge, bge,
                     n0, ns, nr, g0, gb):
    n0v = _dot(x_ref[...], wen[...]) + ben[...]
    n0[...] = n0v
    ns[...] = _dot(n0v, ws[...])
    nr[...] = _dot(n0v, wr[...])
    g0v = _dot(gf[...], weg[...]) + beg[...]
    g0[...] = g0v
    gb[...] = _dot(g0v, wge[...]) + bge[...]


def _embed_nodes(nodes, wen, ben, ws1, wr1, gf, weg, beg, wge1, bge1):
    full = lambda i: (0, 0)
    return pl.pallas_call(
        _embed_node_body,
        grid=(GN,),
        in_specs=[
            pl.BlockSpec((BN, DL), lambda i: (i, 0)),
            pl.BlockSpec((DL, DL), full),
            pl.BlockSpec((1, DL), full),
            pl.BlockSpec((DL, DL), full),
            pl.BlockSpec((DL, DL), full),
            pl.BlockSpec((1, DL), full),
            pl.BlockSpec((DL, DL), full),
            pl.BlockSpec((1, DL), full),
            pl.BlockSpec((DL, DL), full),
            pl.BlockSpec((1, DL), full),
        ],
        out_specs=[
            pl.BlockSpec((BN, DL), lambda i: (i, 0)),
            pl.BlockSpec((BN, DL), lambda i: (i, 0)),
            pl.BlockSpec((BN, DL), lambda i: (i, 0)),
            pl.BlockSpec((1, DL), full),
            pl.BlockSpec((1, DL), full),
        ],
        out_shape=[
            jax.ShapeDtypeStruct((N, DL), jnp.float32),
            jax.ShapeDtypeStruct((N, DL), jnp.float32),
            jax.ShapeDtypeStruct((N, DL), jnp.float32),
            jax.ShapeDtypeStruct((1, DL), jnp.float32),
            jax.ShapeDtypeStruct((1, DL), jnp.float32),
        ],
    )(nodes, wen, ben.reshape(1, DL), ws1, wr1, gf, weg, beg.reshape(1, DL),
      wge1, bge1.reshape(1, DL))


def _edge_body(e_ref, gs_ref, we, geb, lns, lno, new_e, e2, eagg):
    i = pl.program_id(0)
    ev = e_ref[...].astype(jnp.float32)
    x = _dot(ev, we[...]) + gs_ref[...] + geb[...]
    _edge_common(i, ev, x, lns, lno, new_e, e2, eagg)


def _edge_common(i, ev, x, lns, lno, new_e, e2, eagg):
    ne = jnp.maximum(x, 0.0)
    new_e[...] = ne
    e2[...] = _ln(ne + ev, lns[...], lno[...]).astype(e2.dtype)
    part = jnp.sum(ne.reshape(BE // 8, 8, DL), axis=0)

    @pl.when(i == 0)
    def _():
        eagg[...] = part

    @pl.when(i > 0)
    def _():
        eagg[...] = eagg[...] + part


def _edge_embed_body(x_ref, gs_ref, wemb, bemb, we, geb, lns, lno,
                     new_e, e2, eagg):
    i = pl.program_id(0)
    ev = _dot(x_ref[...], wemb[...]) + bemb[...]
    x = _dot(ev, we[...]) + gs_ref[...] + geb[...]
    _edge_common(i, ev, x, lns, lno, new_e, e2, eagg)


def _tc_edge_embed(edges, gsum, wemb, bemb, we_e, geb, ln_scale, ln_offset):
    full = lambda i: (0, 0)
    return pl.pallas_call(
        _edge_embed_body,
        grid=(GE,),
        in_specs=[
            pl.BlockSpec((BE, D_EDGE), lambda i: (i, 0)),
            pl.BlockSpec((BE, DL), lambda i: (i, 0)),
            pl.BlockSpec((D_EDGE, DL), full),
            pl.BlockSpec((1, DL), full),
            pl.BlockSpec((DL, DL), full),
            pl.BlockSpec((1, DL), full),
            pl.BlockSpec((1, DL), full),
            pl.BlockSpec((1, DL), full),
        ],
        out_specs=[
            pl.BlockSpec((BE, DL), lambda i: (i, 0)),
            pl.BlockSpec((BE, DL), lambda i: (i, 0)),
            pl.BlockSpec((8, DL), full),
        ],
        out_shape=[
            jax.ShapeDtypeStruct((E, DL), jnp.float32),
            jax.ShapeDtypeStruct((E, DL), jnp.bfloat16),
            jax.ShapeDtypeStruct((8, DL), jnp.float32),
        ],
    )(edges, gsum, wemb, bemb.reshape(1, DL), we_e, geb,
      ln_scale.reshape(1, DL), ln_offset.reshape(1, DL))


def _tc_edge(e, gsum, we_e, geb, ln_scale, ln_offset):
    full = lambda i: (0, 0)
    return pl.pallas_call(
        _edge_body,
        grid=(GE,),
        in_specs=[
            pl.BlockSpec((BE, DL), lambda i: (i, 0)),
            pl.BlockSpec((BE, DL), lambda i: (i, 0)),
            pl.BlockSpec((DL, DL), full),
            pl.BlockSpec((1, DL), full),
            pl.BlockSpec((1, DL), full),
            pl.BlockSpec((1, DL), full),
        ],
        out_specs=[
            pl.BlockSpec((BE, DL), lambda i: (i, 0)),
            pl.BlockSpec((BE, DL), lambda i: (i, 0)),
            pl.BlockSpec((8, DL), full),
        ],
        out_shape=[
            jax.ShapeDtypeStruct((E, DL), jnp.float32),
            jax.ShapeDtypeStruct((E, DL), jnp.float32),
            jax.ShapeDtypeStruct((8, DL), jnp.float32),
        ],
    )(e, gsum, we_e, geb, ln_scale.reshape(1, DL), ln_offset.reshape(1, DL))


def _make_node_body(last):
    def body(n_ref, sa_ref, ra_ref, g_ref, eagg_ref,
             wn_n, wn_s, wn_r, wn_g, bn, wg_n, wg_e, wg_g, bg,
             lnn_s, lnn_o, lng_s, lng_o, xa, xb, xc, xd,
             n2, o1, o2, o3, o4, nacc):
        i = pl.program_id(0)
        nv = n_ref[...]
        gv = g_ref[...]
        gn = _dot(gv, wn_g[...]) + bn[...]
        x = _dot(nv, wn_n[...]) + _dot(sa_ref[...], wn_s[...]) \
            + _dot(ra_ref[...], wn_r[...]) + gn
        nn = jnp.maximum(x, 0.0)
        n2v = _ln(nn + nv, lnn_s[...], lnn_o[...])
        n2[...] = n2v
        part = jnp.sum(nn.reshape(BN // 8, 8, DL), axis=0)

        @pl.when(i == 0)
        def _():
            nacc[...] = part

        @pl.when(i > 0)
        def _():
            nacc[...] = nacc[...] + part

        if not last:
            o1[...] = _dot(n2v, xa[...])
            o2[...] = _dot(n2v, xb[...])

        @pl.when(i == GN - 1)
        def _():
            nagg = jnp.sum(nacc[...], axis=0, keepdims=True)
            eagg = jnp.sum(eagg_ref[...], axis=0, keepdims=True)
            ng = jnp.maximum(
                _dot(nagg, wg_n[...]) + _dot(eagg, wg_e[...])
                + _dot(gv, wg_g[...]) + bg[...], 0.0)
            g2v = _ln(ng + gv, lng_s[...], lng_o[...])
            if last:
                o1[...] = _dot(g2v, xa[...]) + xb[...]
            else:
                o3[...] = g2v
                o4[...] = _dot(g2v, xc[...]) + xd[...]

    return body


def _tc_node(n, sa, ra, g, eagg8, sp, last, xa, xb, xc=None, xd=None):
    wn = sp["node_mlp"]["w"]
    wg = sp["global_mlp"]["w"]
    full = lambda i: (0, 0)
    blk = lambda i: (i, 0)
    row = pl.BlockSpec((1, DL), full)
    mat = pl.BlockSpec((DL, DL), full)
    nblk = pl.BlockSpec((BN, DL), blk)
    in_specs = [nblk, nblk, nblk, row, pl.BlockSpec((8, DL), full),
                mat, mat, mat, mat, row, mat, mat, mat, row,
                row, row, row, row]
    args = [n, sa, ra, g, eagg8,
            wn[:DL], wn[DL:2 * DL], wn[2 * DL:3 * DL], wn[3 * DL:],
            sp["node_mlp"]["b"].reshape(1, DL),
            wg[:DL], wg[DL:2 * DL], wg[2 * DL:],
            sp["global_mlp"]["b"].reshape(1, DL),
            sp["ln_nodes"]["scale"].reshape(1, DL),
            sp["ln_nodes"]["offset"].reshape(1, DL),
            sp["ln_globals"]["scale"].reshape(1, DL),
            sp["ln_globals"]["offset"].reshape(1, DL)]
    if last:
        in_specs += [mat, row]
        args += [xa, xb.reshape(1, DL)]
        out_specs = [nblk, row]
        out_shape = [jax.ShapeDtypeStruct((N, DL), jnp.float32),
                     jax.ShapeDtypeStruct((1, DL), jnp.float32)]
    else:
        in_specs += [mat, mat, mat, row]
        args += [xa, xb, xc, xd.reshape(1, DL)]
        out_specs = [nblk, nblk, nblk, row, row]
        out_shape = [jax.ShapeDtypeStruct((N, DL), jnp.float32),
                     jax.ShapeDtypeStruct((N, DL), jnp.float32),
                     jax.ShapeDtypeStruct((N, DL), jnp.float32),
                     jax.ShapeDtypeStruct((1, DL), jnp.float32),
                     jax.ShapeDtypeStruct((1, DL), jnp.float32)]
    body = _make_node_body(last)
    if last:
        def wrapped(*refs):
            ins = refs[:20]
            n2, o1 = refs[20:22]
            nacc = refs[22]
            body(*ins[:18], ins[18], ins[19], None, None,
                 n2, o1, None, None, None, nacc)
    else:
        def wrapped(*refs):
            ins = refs[:22]
            n2, o1, o2, o3, o4 = refs[22:27]
            nacc = refs[27]
            body(*ins[:18], ins[18], ins[19], ins[20], ins[21],
                 n2, o1, o2, o3, o4, nacc)
    return pl.pallas_call(
        wrapped,
        grid=(GN,),
        in_specs=in_specs,
        out_specs=out_specs,
        out_shape=out_shape,
        scratch_shapes=[pltpu.VMEM((8, DL), jnp.float32)],
    )(*args)


def _sc_gather(ns, nr, senders, receivers):
    mesh = plsc.VectorSubcoreMesh(core_axis_name="c", subcore_axis_name="s")
    nchunk = EW // CH + 1

    @functools.partial(
        pl.kernel,
        mesh=mesh,
        out_type=jax.ShapeDtypeStruct((E, DL), jnp.float32),
        scratch_types=[
            pltpu.VMEM((EW,), jnp.int32),
            pltpu.VMEM((EW,), jnp.int32),
            pltpu.VMEM((2, CH, DL), jnp.float32),
            pltpu.VMEM((2, CH, DL), jnp.float32),
            pltpu.VMEM((2, CH, DL), jnp.float32),
            pltpu.SemaphoreType.DMA,
            pltpu.SemaphoreType.DMA,
        ],
    )
    def k(ns_h, nr_h, s_h, r_h, out_h, idx_s, idx_r, rows_a, rows_b,
          rows_o, sem0, sem1):
        wid = lax.axis_index("s") * NC + lax.axis_index("c")
        base = wid * EW
        sems = (sem0, sem1)

        pltpu.sync_copy(s_h.at[pl.ds(base, EW)], idx_s)
        pltpu.sync_copy(r_h.at[pl.ds(base, EW)], idx_r)

        def loff(c):
            return jnp.minimum(c * CH, EW - CH)

        def issue(c, b):
            o = loff(c)
            pltpu.async_copy(ns_h.at[idx_s.at[pl.ds(o, CH)]],
                             rows_a.at[b], sems[b])
            pltpu.async_copy(nr_h.at[idx_r.at[pl.ds(o, CH)]],
                             rows_b.at[b], sems[b])

        def drain(c, b):
            o = loff(c)
            pltpu.make_async_copy(ns_h.at[idx_s.at[pl.ds(o, CH)]],
                                  rows_a.at[b], sems[b]).wait()
            pltpu.make_async_copy(nr_h.at[idx_r.at[pl.ds(o, CH)]],
                                  rows_b.at[b], sems[b]).wait()

        def process(c, b):
            def add_row(i, carry):
                for v in range(DL // LANES):
                    sl = pl.ds(v * LANES, LANES)
                    rows_o[b, i, sl] = rows_a[b, i, sl] + rows_b[b, i, sl]
                return carry

            lax.fori_loop(0, CH, add_row, 0)
            pltpu.sync_copy(rows_o.at[b], out_h.at[pl.ds(base + loff(c), CH)])

        issue(0, 0)

        def body(j2, _):
            c0 = j2 * 2
            issue(c0 + 1, 1)
            drain(c0, 0)
            process(c0, 0)
            issue(c0 + 2, 0)
            drain(c0 + 1, 1)
            process(c0 + 1, 1)
            return 0

        lax.fori_loop(0, nchunk // 2, body, 0)
        drain(nchunk, 0)

    return k(ns, nr, senders, receivers)


def _sc_scatter(new_e, sr_flat, zrows):
    mesh = plsc.VectorSubcoreMesh(core_axis_name="c", subcore_axis_name="s")

    nchunk = ET // CH

    @functools.partial(
        pl.kernel,
        mesh=mesh,
        out_type=jax.ShapeDtypeStruct((2 * N, DL), jnp.float32),
        scratch_types=[
            pltpu.VMEM((nchunk, CH), jnp.int32),
            pltpu.VMEM((LANES,), jnp.int32),
            pltpu.VMEM((2, CH, DL), jnp.float32),
            pltpu.VMEM_SHARED((NPAD, DL), jnp.float32),
            pltpu.SemaphoreType.DMA,
            pltpu.SemaphoreType.DMA,
        ],
    )
    def k(ne_h, sr_h, z_h, out_h, idx2d, idx_t, rows, acc, sem0, sem1):
        c = lax.axis_index("c")
        s = lax.axis_index("s")
        base = s * ET
        sems = (sem0, sem1)
        pltpu.sync_copy(z_h, acc.at[pl.ds(s * NT, NT)])
        plsc.subcore_barrier()

        def issue(j, b):
            jc = jnp.minimum(j, nchunk - 1)
            pltpu.async_copy(sr_h.at[pl.ds(c * E + base + jc * CH, CH)],
                             idx2d.at[jc], sems[b])
            pltpu.async_copy(ne_h.at[pl.ds(base + jc * CH, CH)],
                             rows.at[b], sems[b])

        def drain(j, b):
            jc = jnp.minimum(j, nchunk - 1)
            pltpu.make_async_copy(sr_h.at[pl.ds(c * E + base + jc * CH, CH)],
                                  idx2d.at[jc], sems[b]).wait()
            pltpu.make_async_copy(ne_h.at[pl.ds(base + jc * CH, CH)],
                                  rows.at[b], sems[b]).wait()

        def scat(j, b):
            pltpu.sync_copy(rows.at[b], acc.at[idx2d.at[jnp.minimum(j, nchunk - 1)]],
                            add=True)

        issue(0, 0)

        def body(j2, _):
            c0 = j2 * 2
            issue(c0 + 1, 1)
            drain(c0, 0)
            scat(c0, 0)
            issue(c0 + 2, 0)
            drain(c0 + 1, 1)
            scat(c0 + 1, 1)
            return 0

        lax.fori_loop(0, nchunk // 2, body, 0)
        drain(nchunk, 0)
        off = base + nchunk * CH
        pltpu.sync_copy(sr_h.at[pl.ds(c * E + off, LANES)], idx_t)
        pltpu.sync_copy(ne_h.at[pl.ds(off, LANES)],
                        rows.at[0, pl.ds(0, LANES)])
        pltpu.sync_copy(rows.at[0, pl.ds(0, LANES)], acc.at[idx_t], add=True)
        plsc.subcore_barrier()

        @pl.when(s < NS - 1)
        def _():
            pltpu.sync_copy(acc.at[pl.ds(s * NT, NT)],
                            out_h.at[pl.ds(c * N + s * NT, NT)])

        @pl.when(s == NS - 1)
        def _():
            pltpu.sync_copy(acc.at[pl.ds((NS - 1) * NT, N - (NS - 1) * NT)],
                            out_h.at[pl.ds(c * N + (NS - 1) * NT,
                                           N - (NS - 1) * NT)])

    return k(new_e, sr_flat, zrows)


def kernel(nodes, edges, senders, receivers, global_feats, params):
    p = params
    st = p["steps"]

    def edge_w(s):
        w = st[s]["edge_mlp"]["w"]
        return w[:DL], w[DL:2 * DL], w[2 * DL:3 * DL], w[3 * DL:]

    sr_flat = jnp.concatenate([senders, receivers])
    zrows = jnp.zeros((NT, DL), jnp.float32)

    we_e0, we_s0, we_r0, we_g0 = edge_w(0)
    n, ns, nr, g, gb = _embed_nodes(
        nodes, p["embed_node"]["w"], p["embed_node"]["b"], we_s0, we_r0,
        global_feats, p["embed_global"]["w"], p["embed_global"]["b"],
        we_g0, st[0]["edge_mlp"]["b"])

    gdec = None
    for s in range(len(st)):
        last = s == len(st) - 1
        gsum = _sc_gather(ns, nr, senders, receivers)
        if s == 0:
            new_e, e, eagg8 = _tc_edge_embed(
                edges, gsum, p["embed_edge"]["w"], p["embed_edge"]["b"],
                we_e0, gb,
                st[s]["ln_edges"]["scale"], st[s]["ln_edges"]["offset"])
        else:
            new_e, e, eagg8 = _tc_edge(
                e, gsum, edge_w(s)[0], gb,
                st[s]["ln_edges"]["scale"], st[s]["ln_edges"]["offset"])
        aggs = _sc_scatter(new_e, sr_flat, zrows)
        sa, ra = aggs[:N], aggs[N:]
        if last:
            n, gdec = _tc_node(n, sa, ra, g, eagg8, st[s], True,
                               p["decoder"]["w"], p["decoder"]["b"])
        else:
            _, we_s1, we_r1, we_g1 = edge_w(s + 1)
            n, ns, nr, g, gb = _tc_node(n, sa, ra, g, eagg8, st[s], False,
                                        we_s1, we_r1, we_g1,
                                        st[s + 1]["edge_mlp"]["b"])
    return (n, e, gdec)


# speedup vs baseline: 1.1743x; 1.1092x over previous
import functools

import jax
import jax.numpy as jnp
from jax import lax
from jax.experimental import pallas as pl
from jax.experimental.pallas import tpu as pltpu
from jax.experimental.pallas import tpu_sc as plsc

N = 10000
E = 160000
DL = 128
D_EDGE = 16

BE = 8000
GE = E // BE
BN = 5000
GN = N // BN

NC = 2
NS = 16
NW = NC * NS
LANES = 16
CH = 128
EW = E // NW
ET = E // NS
NT = 640
NPAD = NT * NS


def _ln(y, scale, offset):
    mu = jnp.mean(y, axis=-1, keepdims=True)
    var = jnp.mean(jnp.square(y - mu), axis=-1, keepdims=True)
    return (y - mu) * lax.rsqrt(var + 1e-5) * scale + offset


def _dot(a, b):
    return jnp.dot(a, b, preferred_element_type=jnp.float32)


def _embed_node_body(x_ref, wen, ben, ws, wr, gf, weg, beg, wge, bge,
                     n0, ns, nr, g0, gb):
    n0v = _dot(x_ref[...], wen[...]) + ben[...]
    n0[...] = n0v
    ns[...] = _dot(n0v, ws[...])
    nr[...] = _dot(n0v, wr[...])
    g0v = _dot(gf[...], weg[...]) + beg[...]
    g0[...] = g0v
    gb[...] = _dot(g0v, wge[...]) + bge[...]


def _embed_nodes(nodes, wen, ben, ws1, wr1, gf, weg, beg, wge1, bge1):
    full = lambda i: (0, 0)
    return pl.pallas_call(
        _embed_node_body,
        grid=(GN,),
        in_specs=[
            pl.BlockSpec((BN, DL), lambda i: (i, 0)),
            pl.BlockSpec((DL, DL), full),
            pl.BlockSpec((1, DL), full),
            pl.BlockSpec((DL, DL), full),
            pl.BlockSpec((DL, DL), full),
            pl.BlockSpec((1, DL), full),
            pl.BlockSpec((DL, DL), full),
            pl.BlockSpec((1, DL), full),
            pl.BlockSpec((DL, DL), full),
            pl.BlockSpec((1, DL), full),
        ],
        out_specs=[
            pl.BlockSpec((BN, DL), lambda i: (i, 0)),
            pl.BlockSpec((BN, DL), lambda i: (i, 0)),
            pl.BlockSpec((BN, DL), lambda i: (i, 0)),
            pl.BlockSpec((1, DL), full),
            pl.BlockSpec((1, DL), full),
        ],
        out_shape=[
            jax.ShapeDtypeStruct((N, DL), jnp.float32),
            jax.ShapeDtypeStruct((N, DL), jnp.float32),
            jax.ShapeDtypeStruct((N, DL), jnp.float32),
            jax.ShapeDtypeStruct((1, DL), jnp.float32),
            jax.ShapeDtypeStruct((1, DL), jnp.float32),
        ],
    )(nodes, wen, ben.reshape(1, DL), ws1, wr1, gf, weg, beg.reshape(1, DL),
      wge1, bge1.reshape(1, DL))


def _edge_body(e_ref, gs_ref, we, geb, lns, lno, new_e, e2, eagg):
    i = pl.program_id(0)
    ev = e_ref[...].astype(jnp.float32)
    x = _dot(ev, we[...]) + gs_ref[...] + geb[...]
    _edge_common(i, ev, x, lns, lno, new_e, e2, eagg)


def _edge_common(i, ev, x, lns, lno, new_e, e2, eagg):
    ne = jnp.maximum(x, 0.0)
    new_e[...] = ne
    e2[...] = _ln(ne + ev, lns[...], lno[...]).astype(e2.dtype)
    part = jnp.sum(ne.reshape(BE // 8, 8, DL), axis=0)

    @pl.when(i == 0)
    def _():
        eagg[...] = part

    @pl.when(i > 0)
    def _():
        eagg[...] = eagg[...] + part


def _edge_embed_body(x_ref, gs_ref, wemb, bemb, we, geb, lns, lno,
                     new_e, e2, eagg):
    i = pl.program_id(0)
    ev = _dot(x_ref[...], wemb[...]) + bemb[...]
    x = _dot(ev, we[...]) + gs_ref[...] + geb[...]
    _edge_common(i, ev, x, lns, lno, new_e, e2, eagg)


def _tc_edge_embed(edges, gsum, wemb, bemb, we_e, geb, ln_scale, ln_offset):
    full = lambda i: (0, 0)
    return pl.pallas_call(
        _edge_embed_body,
        grid=(GE,),
        in_specs=[
            pl.BlockSpec((BE, D_EDGE), lambda i: (i, 0)),
            pl.BlockSpec((BE, DL), lambda i: (i, 0)),
            pl.BlockSpec((D_EDGE, DL), full),
            pl.BlockSpec((1, DL), full),
            pl.BlockSpec((DL, DL), full),
            pl.BlockSpec((1, DL), full),
            pl.BlockSpec((1, DL), full),
            pl.BlockSpec((1, DL), full),
        ],
        out_specs=[
            pl.BlockSpec((BE, DL), lambda i: (i, 0)),
            pl.BlockSpec((BE, DL), lambda i: (i, 0)),
            pl.BlockSpec((8, DL), full),
        ],
        out_shape=[
            jax.ShapeDtypeStruct((E, DL), jnp.float32),
            jax.ShapeDtypeStruct((E, DL), jnp.bfloat16),
            jax.ShapeDtypeStruct((8, DL), jnp.float32),
        ],
    )(edges, gsum, wemb, bemb.reshape(1, DL), we_e, geb,
      ln_scale.reshape(1, DL), ln_offset.reshape(1, DL))


def _tc_edge(e, gsum, we_e, geb, ln_scale, ln_offset):
    full = lambda i: (0, 0)
    return pl.pallas_call(
        _edge_body,
        grid=(GE,),
        in_specs=[
            pl.BlockSpec((BE, DL), lambda i: (i, 0)),
            pl.BlockSpec((BE, DL), lambda i: (i, 0)),
            pl.BlockSpec((DL, DL), full),
            pl.BlockSpec((1, DL), full),
            pl.BlockSpec((1, DL), full),
            pl.BlockSpec((1, DL), full),
        ],
        out_specs=[
            pl.BlockSpec((BE, DL), lambda i: (i, 0)),
            pl.BlockSpec((BE, DL), lambda i: (i, 0)),
            pl.BlockSpec((8, DL), full),
        ],
        out_shape=[
            jax.ShapeDtypeStruct((E, DL), jnp.float32),
            jax.ShapeDtypeStruct((E, DL), jnp.float32),
            jax.ShapeDtypeStruct((8, DL), jnp.float32),
        ],
    )(e, gsum, we_e, geb, ln_scale.reshape(1, DL), ln_offset.reshape(1, DL))


def _make_node_body(last):
    def body(n_ref, sa_ref, ra_ref, g_ref, eagg_ref,
             wn_n, wn_s, wn_r, wn_g, bn, wg_n, wg_e, wg_g, bg,
             lnn_s, lnn_o, lng_s, lng_o, xa, xb, xc, xd,
             n2, o1, o2, o3, o4, nacc):
        i = pl.program_id(0)
        nv = n_ref[...]
        gv = g_ref[...]
        gn = _dot(gv, wn_g[...]) + bn[...]
        x = _dot(nv, wn_n[...]) + _dot(sa_ref[...], wn_s[...]) \
            + _dot(ra_ref[...], wn_r[...]) + gn
        nn = jnp.maximum(x, 0.0)
        n2v = _ln(nn + nv, lnn_s[...], lnn_o[...])
        n2[...] = n2v
        part = jnp.sum(nn.reshape(BN // 8, 8, DL), axis=0)

        @pl.when(i == 0)
        def _():
            nacc[...] = part

        @pl.when(i > 0)
        def _():
            nacc[...] = nacc[...] + part

        if not last:
            o1[...] = _dot(n2v, xa[...])
            o2[...] = _dot(n2v, xb[...])

        @pl.when(i == GN - 1)
        def _():
            nagg = jnp.sum(nacc[...], axis=0, keepdims=True)
            eagg = jnp.sum(eagg_ref[...], axis=0, keepdims=True)
            ng = jnp.maximum(
                _dot(nagg, wg_n[...]) + _dot(eagg, wg_e[...])
                + _dot(gv, wg_g[...]) + bg[...], 0.0)
            g2v = _ln(ng + gv, lng_s[...], lng_o[...])
            if last:
                o1[...] = _dot(g2v, xa[...]) + xb[...]
            else:
                o3[...] = g2v
                o4[...] = _dot(g2v, xc[...]) + xd[...]

    return body


def _tc_node(n, sa, ra, g, eagg8, sp, last, xa, xb, xc=None, xd=None):
    wn = sp["node_mlp"]["w"]
    wg = sp["global_mlp"]["w"]
    full = lambda i: (0, 0)
    blk = lambda i: (i, 0)
    row = pl.BlockSpec((1, DL), full)
    mat = pl.BlockSpec((DL, DL), full)
    nblk = pl.BlockSpec((BN, DL), blk)
    in_specs = [nblk, nblk, nblk, row, pl.BlockSpec((8, DL), full),
                mat, mat, mat, mat, row, mat, mat, mat, row,
                row, row, row, row]
    args = [n, sa, ra, g, eagg8,
            wn[:DL], wn[DL:2 * DL], wn[2 * DL:3 * DL], wn[3 * DL:],
            sp["node_mlp"]["b"].reshape(1, DL),
            wg[:DL], wg[DL:2 * DL], wg[2 * DL:],
            sp["global_mlp"]["b"].reshape(1, DL),
            sp["ln_nodes"]["scale"].reshape(1, DL),
            sp["ln_nodes"]["offset"].reshape(1, DL),
            sp["ln_globals"]["scale"].reshape(1, DL),
            sp["ln_globals"]["offset"].reshape(1, DL)]
    if last:
        in_specs += [mat, row]
        args += [xa, xb.reshape(1, DL)]
        out_specs = [nblk, row]
        out_shape = [jax.ShapeDtypeStruct((N, DL), jnp.float32),
                     jax.ShapeDtypeStruct((1, DL), jnp.float32)]
    else:
        in_specs += [mat, mat, mat, row]
        args += [xa, xb, xc, xd.reshape(1, DL)]
        out_specs = [nblk, nblk, nblk, row, row]
        out_shape = [jax.ShapeDtypeStruct((N, DL), jnp.float32),
                     jax.ShapeDtypeStruct((N, DL), jnp.float32),
                     jax.ShapeDtypeStruct((N, DL), jnp.float32),
                     jax.ShapeDtypeStruct((1, DL), jnp.float32),
                     jax.ShapeDtypeStruct((1, DL), jnp.float32)]
    body = _make_node_body(last)
    if last:
        def wrapped(*refs):
            ins = refs[:20]
            n2, o1 = refs[20:22]
            nacc = refs[22]
            body(*ins[:18], ins[18], ins[19], None, None,
                 n2, o1, None, None, None, nacc)
    else:
        def wrapped(*refs):
            ins = refs[:22]
            n2, o1, o2, o3, o4 = refs[22:27]
            nacc = refs[27]
            body(*ins[:18], ins[18], ins[19], ins[20], ins[21],
                 n2, o1, o2, o3, o4, nacc)
    return pl.pallas_call(
        wrapped,
        grid=(GN,),
        in_specs=in_specs,
        out_specs=out_specs,
        out_shape=out_shape,
        scratch_shapes=[pltpu.VMEM((8, DL), jnp.float32)],
    )(*args)


def _sc_gather(ns, nr, senders, receivers):
    mesh = plsc.VectorSubcoreMesh(core_axis_name="c", subcore_axis_name="s")
    nchunk = EW // CH + 1

    @functools.partial(
        pl.kernel,
        mesh=mesh,
        out_type=jax.ShapeDtypeStruct((E, DL), jnp.float32),
        scratch_types=[
            pltpu.VMEM((EW,), jnp.int32),
            pltpu.VMEM((EW,), jnp.int32),
            pltpu.VMEM((2, CH, DL), jnp.float32),
            pltpu.VMEM((2, CH, DL), jnp.float32),
            pltpu.VMEM((2, CH, DL), jnp.float32),
            pltpu.SemaphoreType.DMA,
            pltpu.SemaphoreType.DMA,
            pltpu.SemaphoreType.DMA,
            pltpu.SemaphoreType.DMA,
        ],
    )
    def k(ns_h, nr_h, s_h, r_h, out_h, idx_s, idx_r, rows_a, rows_b,
          rows_o, sem0, sem1, wsem0, wsem1):
        wid = lax.axis_index("s") * NC + lax.axis_index("c")
        base = wid * EW
        sems = (sem0, sem1)
        wsems = (wsem0, wsem1)

        pltpu.sync_copy(s_h.at[pl.ds(base, EW)], idx_s)
        pltpu.sync_copy(r_h.at[pl.ds(base, EW)], idx_r)

        def loff(c):
            return jnp.minimum(c * CH, EW - CH)

        def issue(c, b):
            o = loff(c)
            pltpu.async_copy(ns_h.at[idx_s.at[pl.ds(o, CH)]],
                             rows_a.at[b], sems[b])
            pltpu.async_copy(nr_h.at[idx_r.at[pl.ds(o, CH)]],
                             rows_b.at[b], sems[b])

        def drain(c, b):
            o = loff(c)
            pltpu.make_async_copy(ns_h.at[idx_s.at[pl.ds(o, CH)]],
                                  rows_a.at[b], sems[b]).wait()
            pltpu.make_async_copy(nr_h.at[idx_r.at[pl.ds(o, CH)]],
                                  rows_b.at[b], sems[b]).wait()

        def wait_store(c, b):
            pltpu.make_async_copy(rows_o.at[b],
                                  out_h.at[pl.ds(base + loff(c), CH)],
                                  wsems[b]).wait()

        def process(c, b):
            def add_row(i, carry):
                for v in range(DL // LANES):
                    sl = pl.ds(v * LANES, LANES)
                    rows_o[b, i, sl] = rows_a[b, i, sl] + rows_b[b, i, sl]
                return carry

            lax.fori_loop(0, CH, add_row, 0)
            pltpu.async_copy(rows_o.at[b],
                             out_h.at[pl.ds(base + loff(c), CH)], wsems[b])

        issue(0, 0)

        def body(j2, _):
            c0 = j2 * 2
            issue(c0 + 1, 1)
            drain(c0, 0)

            @pl.when(j2 > 0)
            def _():
                wait_store(c0 - 2, 0)

            process(c0, 0)
            issue(c0 + 2, 0)
            drain(c0 + 1, 1)

            @pl.when(j2 > 0)
            def _():
                wait_store(c0 - 1, 1)

            process(c0 + 1, 1)
            return 0

        lax.fori_loop(0, nchunk // 2, body, 0)
        drain(nchunk, 0)
        wait_store(nchunk - 2, 0)
        wait_store(nchunk - 1, 1)

    return k(ns, nr, senders, receivers)


def _sc_scatter(new_e, sr_flat, zrows):
    mesh = plsc.VectorSubcoreMesh(core_axis_name="c", subcore_axis_name="s")

    nchunk = ET // CH
    nloop = (nchunk - 2) // 4 * 4

    @functools.partial(
        pl.kernel,
        mesh=mesh,
        out_type=jax.ShapeDtypeStruct((2 * N, DL), jnp.float32),
        scratch_types=[
            pltpu.VMEM((nchunk, CH), jnp.int32),
            pltpu.VMEM((LANES,), jnp.int32),
            pltpu.VMEM((4, CH, DL), jnp.float32),
            pltpu.VMEM_SHARED((NPAD, DL), jnp.float32),
            pltpu.SemaphoreType.DMA,
            pltpu.SemaphoreType.DMA,
            pltpu.SemaphoreType.DMA,
            pltpu.SemaphoreType.DMA,
            pltpu.SemaphoreType.DMA,
            pltpu.SemaphoreType.DMA,
            pltpu.SemaphoreType.DMA,
            pltpu.SemaphoreType.DMA,
        ],
    )
    def k(ne_h, sr_h, z_h, out_h, idx2d, idx_t, rows, acc,
          ls0, ls1, ls2, ls3, ss0, ss1, ss2, ss3):
        lsems = (ls0, ls1, ls2, ls3)
        ssems = (ss0, ss1, ss2, ss3)
        c = lax.axis_index("c")
        s = lax.axis_index("s")
        base = s * ET
        pltpu.sync_copy(z_h, acc.at[pl.ds(s * NT, NT)])
        plsc.subcore_barrier()

        def issue(j, b):
            jc = jnp.minimum(j, nchunk - 1)
            pltpu.async_copy(sr_h.at[pl.ds(c * E + base + jc * CH, CH)],
                             idx2d.at[jc], lsems[b])
            pltpu.async_copy(ne_h.at[pl.ds(base + jc * CH, CH)],
                             rows.at[b], lsems[b])

        def drain(j, b):
            jc = jnp.minimum(j, nchunk - 1)
            pltpu.make_async_copy(sr_h.at[pl.ds(c * E + base + jc * CH, CH)],
                                  idx2d.at[jc], lsems[b]).wait()
            pltpu.make_async_copy(ne_h.at[pl.ds(base + jc * CH, CH)],
                                  rows.at[b], lsems[b]).wait()

        def scat(j, b):
            pltpu.async_copy(rows.at[b],
                             acc.at[idx2d.at[jnp.minimum(j, nchunk - 1)]],
                             ssems[b], add=True)

        def wait_scat(j, b):
            pltpu.make_async_copy(rows.at[b],
                                  acc.at[idx2d.at[jnp.minimum(j, nchunk - 1)]],
                                  ssems[b]).wait()

        issue(0, 0)
        issue(1, 1)

        def body(j4, _):
            c0 = j4 * 4
            for kk in range(4):
                cc = c0 + kk
                b2 = (kk + 2) % 4
                if kk >= 2:
                    wait_scat(cc - 2, b2)
                else:
                    @pl.when(j4 > 0)
                    def _():
                        wait_scat(cc - 2, b2)

                issue(cc + 2, b2)
                drain(cc, kk)
                scat(cc, kk)
            return 0

        lax.fori_loop(0, nloop // 4, body, 0)
        wait_scat(nloop - 2, (nloop - 2) % 4)
        wait_scat(nloop - 1, (nloop - 1) % 4)
        for cc in (nloop, nloop + 1):
            drain(cc, cc % 4)
            scat(cc, cc % 4)
            wait_scat(cc, cc % 4)
        off = base + nchunk * CH
        pltpu.sync_copy(sr_h.at[pl.ds(c * E + off, LANES)], idx_t)
        pltpu.sync_copy(ne_h.at[pl.ds(off, LANES)],
                        rows.at[0, pl.ds(0, LANES)])
        pltpu.sync_copy(rows.at[0, pl.ds(0, LANES)], acc.at[idx_t], add=True)
        plsc.subcore_barrier()

        @pl.when(s < NS - 1)
        def _():
            pltpu.sync_copy(acc.at[pl.ds(s * NT, NT)],
                            out_h.at[pl.ds(c * N + s * NT, NT)])

        @pl.when(s == NS - 1)
        def _():
            pltpu.sync_copy(acc.at[pl.ds((NS - 1) * NT, N - (NS - 1) * NT)],
                            out_h.at[pl.ds(c * N + (NS - 1) * NT,
                                           N - (NS - 1) * NT)])

    return k(new_e, sr_flat, zrows)


def kernel(nodes, edges, senders, receivers, global_feats, params):
    p = params
    st = p["steps"]

    def edge_w(s):
        w = st[s]["edge_mlp"]["w"]
        return w[:DL], w[DL:2 * DL], w[2 * DL:3 * DL], w[3 * DL:]

    sr_flat = jnp.concatenate([senders, receivers])
    zrows = jnp.zeros((NT, DL), jnp.float32)

    we_e0, we_s0, we_r0, we_g0 = edge_w(0)
    n, ns, nr, g, gb = _embed_nodes(
        nodes, p["embed_node"]["w"], p["embed_node"]["b"], we_s0, we_r0,
        global_feats, p["embed_global"]["w"], p["embed_global"]["b"],
        we_g0, st[0]["edge_mlp"]["b"])

    gdec = None
    for s in range(len(st)):
        last = s == len(st) - 1
        gsum = _sc_gather(ns, nr, senders, receivers)
        if s == 0:
            new_e, e, eagg8 = _tc_edge_embed(
                edges, gsum, p["embed_edge"]["w"], p["embed_edge"]["b"],
                we_e0, gb,
                st[s]["ln_edges"]["scale"], st[s]["ln_edges"]["offset"])
        else:
            new_e, e, eagg8 = _tc_edge(
                e, gsum, edge_w(s)[0], gb,
                st[s]["ln_edges"]["scale"], st[s]["ln_edges"]["offset"])
        aggs = _sc_scatter(new_e, sr_flat, zrows)
        sa, ra = aggs[:N], aggs[N:]
        if last:
            n, gdec = _tc_node(n, sa, ra, g, eagg8, st[s], True,
                               p["decoder"]["w"], p["decoder"]["b"])
        else:
            _, we_s1, we_r1, we_g1 = edge_w(s + 1)
            n, ns, nr, g, gb = _tc_node(n, sa, ra, g, eagg8, st[s], False,
                                        we_s1, we_r1, we_g1,
                                        st[s + 1]["edge_mlp"]["b"])
    return (n, e, gdec)
